# baseline (speedup 1.0000x reference)
import functools

import jax
import jax.numpy as jnp
from jax import lax
from jax.experimental import pallas as pl
from jax.experimental.pallas import tpu as pltpu
from jax.experimental.pallas import tpu_sc as plsc

_NUM_CORES = 2
_NUM_SUBCORES = 16
_NUM_WORKERS = _NUM_CORES * _NUM_SUBCORES


def _sc_pool(xf, lengths, tpair, B, S, D):
    D2 = 2 * D
    rpw = B // _NUM_WORKERS
    nvec = D // 16
    ch = 40
    nch_max = S // ch
    chpad = 48
    nvi = chpad // 16
    nbuf = 3

    mesh = plsc.VectorSubcoreMesh(core_axis_name="c", subcore_axis_name="s")

    @functools.partial(
        pl.kernel,
        mesh=mesh,
        out_type=jax.ShapeDtypeStruct((B, D2), jnp.float32),
        scratch_types=[
            pltpu.VMEM((rpw * S + chpad,), jnp.int32),
            pltpu.VMEM((rpw + 16,), jnp.int32),
            pltpu.VMEM((nbuf, nch_max, chpad), jnp.int32),
            pltpu.VMEM((nbuf, S, D2), jnp.float32),
            pltpu.VMEM((rpw, D2), jnp.float32),
        ]
        + [pltpu.SemaphoreType.DMA] * nbuf,
    )
    def k(xf_hbm, len_hbm, tp_hbm, out_hbm, xv, lenv, pidxv, rowsv, repv, *sems):
        wid = lax.axis_index("s") * _NUM_CORES + lax.axis_index("c")
        base = wid * rpw
        pltpu.sync_copy(xf_hbm.at[pl.ds(base * S, rpw * S)], xv.at[pl.ds(0, rpw * S)])
        pltpu.sync_copy(len_hbm.at[pl.ds(base, rpw)], lenv.at[pl.ds(0, rpw)])

        def nchunks(r):
            l = lenv[pl.ds(r, 16)][0]
            return l, (l + (ch - 1)) // ch

        def fire(r, k_buf):
            _, nch = nchunks(r)
            for c in range(nch_max):

                @pl.when(c < nch)
                def _():
                    for v in range(nvi):
                        pidxv[k_buf, c, pl.ds(v * 16, 16)] = (
                            xv[pl.ds(r * S + c * ch + v * 16, 16)] >> 1
                        )
                    pltpu.async_copy(
                        tp_hbm.at[pidxv.at[k_buf, c, pl.ds(0, ch)]],
                        rowsv.at[k_buf, pl.ds(c * ch, ch)],
                        sems[k_buf],
                    )

        def drain(r, k_buf):
            _, nch = nchunks(r)
            for c in range(nch_max):

                @pl.when(c < nch)
                def _():
                    pltpu.make_async_copy(
                        tp_hbm.at[pl.ds(0, ch)],
                        rowsv.at[k_buf, pl.ds(c * ch, ch)],
                        sems[k_buf],
                    ).wait()

        def accumulate(r, k_buf):
            l, nch = nchunks(r)

            def chunk_body(c, accs):
                j0 = c * ch
                for jj in range(ch):
                    j = j0 + jj
                    take = j < l
                    half = (xv[pl.ds(r * S + j, 16)][0] & 1) << 6
                    accs = tuple(
                        accs[q]
                        + jnp.where(
                            take,
                            rowsv[k_buf, j, pl.ds(half + q * 16, 16)],
                            0.0,
                        )
                        for q in range(nvec)
                    )
                return accs

            accs = tuple(jnp.zeros((16,), jnp.float32) for _ in range(nvec))
            accs = lax.fori_loop(0, nch, chunk_body, accs)
            inv = 1.0 / jnp.full((16,), l, jnp.float32)
            for q in range(nvec):
                repv[r, pl.ds(q * 16, 16)] = accs[q] * inv

        for k_buf in range(nbuf):
            fire(k_buf, k_buf)

        def body(i, carry):
            for k_buf in range(nbuf):
                r = i * nbuf + k_buf
                drain(r, k_buf)
                accumulate(r, k_buf)
                nxt = r + nbuf

                @pl.when(nxt < rpw)
                def _():
                    fire(nxt, k_buf)

            return carry

        lax.fori_loop(0, rpw // nbuf, body, 0)

        @pl.when(rpw % nbuf != 0)
        def _():
            for k_buf in range(rpw % nbuf):
                r = (rpw // nbuf) * nbuf + k_buf
                drain(r, k_buf)
                accumulate(r, k_buf)

        pltpu.sync_copy(repv, out_hbm.at[pl.ds(base, rpw)])

    return k(xf, lengths, tpair)


def _tc_head(rep, W, b2, D):
    B, _ = rep.shape
    C = W.shape[1]

    def body(rep_ref, w_ref, b_ref, o_ref):
        r = jnp.maximum(rep_ref[:, :D], 0.0)
        o_ref[...] = (
            lax.dot_general(
                r, w_ref[...], (((1,), (0,)), ((), ())),
                preferred_element_type=jnp.float32,
            )
            + b_ref[...]
        )

    return pl.pallas_call(
        body,
        out_shape=jax.ShapeDtypeStruct((B, C), jnp.float32),
    )(rep, W, b2)


def kernel(x, lengths, table, W, b):
    x = x.astype(jnp.int32)
    lengths = lengths.astype(jnp.int32)
    B, S = x.shape
    V, D = table.shape
    xf = x.reshape(B * S)
    tpair = table.reshape(V // 2, 2 * D)
    rep = _sc_pool(xf, lengths, tpair, B, S, D)
    return _tc_head(rep, W, b.reshape(1, -1), D)
